# baseline (speedup 1.0000x reference)
import functools

import jax
import jax.numpy as jnp
from jax.experimental import pallas as pl
from jax.experimental.pallas import tpu as pltpu
from jax.experimental.pallas import tpu_sc as plsc

_B, _S, _D, _H = 2, 2048, 2048, 16
_HD = _D // _H
_EPS = 1e-5
_SCALE = 1.0 / (_HD ** 0.5)
_NEG = -1e9
_TSN = 512
_TSQ = 1024
_TNQ = 512
_TQ = 1024
_TSO = 1024
_TN = 512
_NT = (((1,), (1,)), ((), ()))
_NN = (((1,), (0,)), ((), ()))


_ROWS = _B * _S
_NW = 32
_RPW = _ROWS // _NW
_CHUNK = 32


def _sc_gather_kernel(table_hbm, idx_hbm, out_hbm, idx_v, rows_v, sem):
    wid = jax.lax.axis_index("s") * 2 + jax.lax.axis_index("c")
    base = wid * _RPW
    for c in range(_RPW // _CHUNK):
        off = base + c * _CHUNK
        pltpu.sync_copy(idx_hbm.at[pl.ds(off, _CHUNK)], idx_v)
        pltpu.async_copy(table_hbm.at[idx_v], rows_v, sem).wait()
        pltpu.sync_copy(rows_v, out_hbm.at[pl.ds(off, _CHUNK)])


def _sc_gather(table, idx):
    mesh = plsc.VectorSubcoreMesh(core_axis_name="c", subcore_axis_name="s",
                                  num_cores=2, num_subcores=16)
    return pl.kernel(
        _sc_gather_kernel,
        out_type=jax.ShapeDtypeStruct((_ROWS, _D), jnp.float32),
        mesh=mesh,
        scratch_types=[
            pltpu.VMEM((_CHUNK,), jnp.int32),
            pltpu.VMEM((_CHUNK, _D), jnp.float32),
            pltpu.SemaphoreType.DMA,
        ],
    )(table, idx)


def _norm_kernel(x_ref, m_ref, n0_ref, n1_ref, g0_ref, g1_ref):
    x = x_ref[0]
    r = jax.lax.rsqrt(jnp.mean(x * x, axis=1, keepdims=True) + _EPS)
    m1 = jnp.max(m_ref[0], axis=1, keepdims=True)
    h0 = x * (r * (1.0 - m1))
    h1 = x * (r * m1)
    g0_ref[0] = (h0 * n0_ref[...]).astype(jnp.bfloat16)
    g1_ref[0] = (h1 * n1_ref[...]).astype(jnp.bfloat16)


def _proj_kernel(g0_ref, g1_ref, w0_ref, w1_ref, ca_ref, sb_ref, o_ref):
    g0 = g0_ref[0]
    g1 = g1_ref[0]
    y = (jax.lax.dot_general(g0, w0_ref[...], _NT,
                             preferred_element_type=jnp.float32)
         + jax.lax.dot_general(g1, w1_ref[...], _NT,
                               preferred_element_type=jnp.float32))
    ia = jax.lax.broadcasted_iota(jnp.int32, (_TNQ, _TNQ), 0)
    ib = jax.lax.broadcasted_iota(jnp.int32, (_TNQ, _TNQ), 1)
    pmat = ((ia ^ 1) == ib).astype(jnp.bfloat16)
    sw = jax.lax.dot_general(y.astype(jnp.bfloat16), pmat, _NN,
                             preferred_element_type=jnp.float32)
    ca = jnp.concatenate([ca_ref[0]] * (_TNQ // _HD), axis=1)
    sb = jnp.concatenate([sb_ref[0]] * (_TNQ // _HD), axis=1)
    o_ref[0] = (y * ca + sw * sb).astype(jnp.bfloat16)


def _attn_kernel(it_ref, jt_ref, q_ref, k_ref, v_ref, o_ref,
                 acc_ref, m_ref, l_ref):
    t = pl.program_id(2)
    i = it_ref[t]
    j = jt_ref[t]
    q = q_ref[0]
    k = k_ref[0]
    s = jax.lax.dot_general(q, k, _NT, preferred_element_type=jnp.float32)
    row = jax.lax.broadcasted_iota(jnp.int32, (_TQ, _TQ), 0)
    col = jax.lax.broadcasted_iota(jnp.int32, (_TQ, _TQ), 1)
    s = jnp.where(jnp.logical_and(i == j, col > row), _NEG, s)
    first = j == 0
    m_prev = jnp.where(first, -3e38,
                       jnp.max(m_ref[...], axis=1, keepdims=True))
    l_prev = jnp.where(first, 0.0,
                       jnp.max(l_ref[...], axis=1, keepdims=True))
    acc_prev = jnp.where(first, 0.0, acc_ref[...])
    m_new = jnp.maximum(m_prev, jnp.max(s, axis=1, keepdims=True))
    p = jnp.exp(s - m_new)
    corr = jnp.exp(m_prev - m_new)
    l_new = l_prev * corr + jnp.sum(p, axis=1, keepdims=True)
    pv = jax.lax.dot_general(p.astype(jnp.bfloat16), v_ref[0], _NN,
                             preferred_element_type=jnp.float32)
    acc_new = acc_prev * corr + pv
    m_ref[...] = jnp.broadcast_to(m_new, (_TQ, 128))
    l_ref[...] = jnp.broadcast_to(l_new, (_TQ, 128))
    acc_ref[...] = acc_new

    @pl.when(i == j)
    def _():
        o_ref[0] = (acc_new / l_new).astype(jnp.bfloat16)


def _out_kernel(a_ref, m_ref, wo0_ref, wo1_ref, o_ref):
    a = a_ref[0]
    m1 = jnp.max(m_ref[0], axis=1, keepdims=True)
    a1 = a * m1.astype(jnp.bfloat16)
    a0 = a - a1
    w0 = wo0_ref[...].astype(jnp.bfloat16)
    w1 = wo1_ref[...].astype(jnp.bfloat16)
    o_ref[0] = (jax.lax.dot_general(a0, w0, _NT,
                                    preferred_element_type=jnp.float32)
                + jax.lax.dot_general(a1, w1, _NT,
                                      preferred_element_type=jnp.float32))


def kernel(x, wq0, wk0, wv0, wo0, wq1, wk1, wv1, wo1, nrm0, nrm1,
           freqs_cos, freqs_sin, mask, modality_ids, start_pos):
    del mask, start_pos
    f32 = jnp.float32
    bf16 = jnp.bfloat16
    m_bc = jnp.broadcast_to(
        (modality_ids == 1).astype(f32)[:, :, None], (_B, _S, 128))
    cc = jnp.repeat(freqs_cos, 2, axis=1)
    sign = jnp.tile(jnp.array([-1.0, 1.0], dtype=f32), _HD // 2)
    ss = jnp.repeat(freqs_sin, 2, axis=1) * sign[None, :]
    ca = jnp.stack([cc * _SCALE, cc, jnp.ones_like(cc)])
    sb = jnp.stack([ss * _SCALE, ss, jnp.zeros_like(ss)])
    n0 = nrm0.reshape(1, _D)
    n1 = nrm1.reshape(1, _D)
    wqkv0 = jnp.concatenate([wq0, wk0, wv0], axis=0).astype(bf16)
    wqkv1 = jnp.concatenate([wq1, wk1, wv1], axis=0).astype(bf16)

    idx = jnp.arange(_ROWS, dtype=jnp.int32)
    x = _sc_gather(x.reshape(_ROWS, _D), idx).reshape(_B, _S, _D)

    g0, g1 = pl.pallas_call(
        _norm_kernel,
        grid=(_B, _S // _TSN),
        in_specs=[
            pl.BlockSpec((1, _TSN, _D), lambda b, i: (b, i, 0)),
            pl.BlockSpec((1, _TSN, 128), lambda b, i: (b, i, 0)),
            pl.BlockSpec((1, _D), lambda b, i: (0, 0)),
            pl.BlockSpec((1, _D), lambda b, i: (0, 0)),
        ],
        out_specs=[
            pl.BlockSpec((1, _TSN, _D), lambda b, i: (b, i, 0)),
            pl.BlockSpec((1, _TSN, _D), lambda b, i: (b, i, 0)),
        ],
        out_shape=[
            jax.ShapeDtypeStruct((_B, _S, _D), bf16),
            jax.ShapeDtypeStruct((_B, _S, _D), bf16),
        ],
    )(x, m_bc, n0, n1)

    nblk = _TNQ // _HD
    qkv = pl.pallas_call(
        _proj_kernel,
        grid=(_B, _S // _TSQ, 3 * _D // _TNQ),
        in_specs=[
            pl.BlockSpec((1, _TSQ, _D), lambda b, i, n: (b, i, 0)),
            pl.BlockSpec((1, _TSQ, _D), lambda b, i, n: (b, i, 0)),
            pl.BlockSpec((_TNQ, _D), lambda b, i, n: (n, 0)),
            pl.BlockSpec((_TNQ, _D), lambda b, i, n: (n, 0)),
            pl.BlockSpec((1, _TSQ, _HD), lambda b, i, n: (n // 4, i, 0)),
            pl.BlockSpec((1, _TSQ, _HD), lambda b, i, n: (n // 4, i, 0)),
        ],
        out_specs=pl.BlockSpec((1, _TSQ, _TNQ), lambda b, i, n: (b, i, n)),
        out_shape=jax.ShapeDtypeStruct((_B, _S, 3 * _D), bf16),
    )(g0, g1, wqkv0, wqkv1, ca, sb)

    nq = _S // _TQ
    tri = [(i, j) for i in range(nq) for j in range(i + 1)]
    itab = jnp.array([i for i, _ in tri], dtype=jnp.int32)
    jtab = jnp.array([j for _, j in tri], dtype=jnp.int32)
    attn = pl.pallas_call(
        _attn_kernel,
        grid_spec=pltpu.PrefetchScalarGridSpec(
            num_scalar_prefetch=2,
            grid=(_B, _H, len(tri)),
            in_specs=[
                pl.BlockSpec((1, _TQ, _HD),
                             lambda b, h, t, it, jt: (b, it[t], h)),
                pl.BlockSpec((1, _TQ, _HD),
                             lambda b, h, t, it, jt: (b, jt[t], _H + h)),
                pl.BlockSpec((1, _TQ, _HD),
                             lambda b, h, t, it, jt: (b, jt[t], 2 * _H + h)),
            ],
            out_specs=pl.BlockSpec((1, _TQ, _HD),
                                   lambda b, h, t, it, jt: (b, it[t], h)),
            scratch_shapes=[
                pltpu.VMEM((_TQ, _HD), jnp.float32),
                pltpu.VMEM((_TQ, 128), jnp.float32),
                pltpu.VMEM((_TQ, 128), jnp.float32),
            ],
        ),
        out_shape=jax.ShapeDtypeStruct((_B, _S, _D), bf16),
    )(itab, jtab, qkv, qkv, qkv)

    out = pl.pallas_call(
        _out_kernel,
        grid=(_B, _S // _TSO, _D // _TN),
        in_specs=[
            pl.BlockSpec((1, _TSO, _D), lambda b, i, n: (b, i, 0)),
            pl.BlockSpec((1, _TSO, 128), lambda b, i, n: (b, i, 0)),
            pl.BlockSpec((_TN, _D), lambda b, i, n: (n, 0)),
            pl.BlockSpec((_TN, _D), lambda b, i, n: (n, 0)),
        ],
        out_specs=pl.BlockSpec((1, _TSO, _TN), lambda b, i, n: (b, i, n)),
        out_shape=jax.ShapeDtypeStruct((_B, _S, _D), jnp.float32),
    )(attn, m_bc, wo0, wo1)

    return out


# speedup vs baseline: 1.0567x; 1.0567x over previous
import jax
import jax.numpy as jnp
from jax.experimental import pallas as pl
from jax.experimental.pallas import tpu as pltpu

_B, _S, _D, _H = 2, 2048, 2048, 16
_HD = _D // _H
_EPS = 1e-5
_SCALE = 1.0 / (_HD ** 0.5)
_NEG = -1e9
_TSN = 512
_TSQ = 1024
_TNQ = 512
_TQ = 1024
_TSO = 1024
_TN = 512
_NT = (((1,), (1,)), ((), ()))
_NN = (((1,), (0,)), ((), ()))


def _norm_kernel(x_ref, m_ref, n0_ref, n1_ref, g0_ref, g1_ref):
    x = x_ref[0]
    r = jax.lax.rsqrt(jnp.mean(x * x, axis=1, keepdims=True) + _EPS)
    m1 = jnp.max(m_ref[0], axis=1, keepdims=True)
    h0 = x * (r * (1.0 - m1))
    h1 = x * (r * m1)
    g0_ref[0] = (h0 * n0_ref[...]).astype(jnp.bfloat16)
    g1_ref[0] = (h1 * n1_ref[...]).astype(jnp.bfloat16)


def _proj_kernel(g0_ref, g1_ref, w0_ref, w1_ref, ca_ref, sb_ref, o_ref):
    g0 = g0_ref[0]
    g1 = g1_ref[0]
    y = (jax.lax.dot_general(g0, w0_ref[...], _NT,
                             preferred_element_type=jnp.float32)
         + jax.lax.dot_general(g1, w1_ref[...], _NT,
                               preferred_element_type=jnp.float32))
    ia = jax.lax.broadcasted_iota(jnp.int32, (_TNQ, _TNQ), 0)
    ib = jax.lax.broadcasted_iota(jnp.int32, (_TNQ, _TNQ), 1)
    pmat = ((ia ^ 1) == ib).astype(jnp.bfloat16)
    sw = jax.lax.dot_general(y.astype(jnp.bfloat16), pmat, _NN,
                             preferred_element_type=jnp.float32)
    ca = jnp.concatenate([ca_ref[0]] * (_TNQ // _HD), axis=1)
    sb = jnp.concatenate([sb_ref[0]] * (_TNQ // _HD), axis=1)
    o_ref[0] = (y * ca + sw * sb).astype(jnp.bfloat16)


def _attn_kernel(it_ref, jt_ref, q_ref, k_ref, v_ref, o_ref,
                 acc_ref, m_ref, l_ref):
    t = pl.program_id(2)
    i = it_ref[t]
    j = jt_ref[t]
    q = q_ref[0]
    k = k_ref[0]
    s = jax.lax.dot_general(q, k, _NT, preferred_element_type=jnp.float32)
    row = jax.lax.broadcasted_iota(jnp.int32, (_TQ, _TQ), 0)
    col = jax.lax.broadcasted_iota(jnp.int32, (_TQ, _TQ), 1)
    s = jnp.where(jnp.logical_and(i == j, col > row), _NEG, s)
    first = j == 0
    m_prev = jnp.where(first, -3e38,
                       jnp.max(m_ref[...], axis=1, keepdims=True))
    l_prev = jnp.where(first, 0.0,
                       jnp.max(l_ref[...], axis=1, keepdims=True))
    acc_prev = jnp.where(first, 0.0, acc_ref[...])
    m_new = jnp.maximum(m_prev, jnp.max(s, axis=1, keepdims=True))
    p = jnp.exp(s - m_new)
    corr = jnp.exp(m_prev - m_new)
    l_new = l_prev * corr + jnp.sum(p, axis=1, keepdims=True)
    pv = jax.lax.dot_general(p.astype(jnp.bfloat16), v_ref[0], _NN,
                             preferred_element_type=jnp.float32)
    acc_new = acc_prev * corr + pv
    m_ref[...] = jnp.broadcast_to(m_new, (_TQ, 128))
    l_ref[...] = jnp.broadcast_to(l_new, (_TQ, 128))
    acc_ref[...] = acc_new

    @pl.when(i == j)
    def _():
        o_ref[0] = (acc_new / l_new).astype(jnp.bfloat16)


def _out_kernel(a_ref, m_ref, wo0_ref, wo1_ref, o_ref):
    a = a_ref[0]
    m1 = jnp.max(m_ref[0], axis=1, keepdims=True)
    a1 = a * m1.astype(jnp.bfloat16)
    a0 = a - a1
    w0 = wo0_ref[...].astype(jnp.bfloat16)
    w1 = wo1_ref[...].astype(jnp.bfloat16)
    o_ref[0] = (jax.lax.dot_general(a0, w0, _NT,
                                    preferred_element_type=jnp.float32)
                + jax.lax.dot_general(a1, w1, _NT,
                                      preferred_element_type=jnp.float32))


def kernel(x, wq0, wk0, wv0, wo0, wq1, wk1, wv1, wo1, nrm0, nrm1,
           freqs_cos, freqs_sin, mask, modality_ids, start_pos):
    del mask, start_pos
    f32 = jnp.float32
    bf16 = jnp.bfloat16
    m_bc = jnp.broadcast_to(
        (modality_ids == 1).astype(f32)[:, :, None], (_B, _S, 128))
    cc = jnp.repeat(freqs_cos, 2, axis=1)
    sign = jnp.tile(jnp.array([-1.0, 1.0], dtype=f32), _HD // 2)
    ss = jnp.repeat(freqs_sin, 2, axis=1) * sign[None, :]
    ca = jnp.stack([cc * _SCALE, cc, jnp.ones_like(cc)])
    sb = jnp.stack([ss * _SCALE, ss, jnp.zeros_like(ss)])
    n0 = nrm0.reshape(1, _D)
    n1 = nrm1.reshape(1, _D)
    wqkv0 = jnp.concatenate([wq0, wk0, wv0], axis=0).astype(bf16)
    wqkv1 = jnp.concatenate([wq1, wk1, wv1], axis=0).astype(bf16)

    g0, g1 = pl.pallas_call(
        _norm_kernel,
        grid=(_B, _S // _TSN),
        in_specs=[
            pl.BlockSpec((1, _TSN, _D), lambda b, i: (b, i, 0)),
            pl.BlockSpec((1, _TSN, 128), lambda b, i: (b, i, 0)),
            pl.BlockSpec((1, _D), lambda b, i: (0, 0)),
            pl.BlockSpec((1, _D), lambda b, i: (0, 0)),
        ],
        out_specs=[
            pl.BlockSpec((1, _TSN, _D), lambda b, i: (b, i, 0)),
            pl.BlockSpec((1, _TSN, _D), lambda b, i: (b, i, 0)),
        ],
        out_shape=[
            jax.ShapeDtypeStruct((_B, _S, _D), bf16),
            jax.ShapeDtypeStruct((_B, _S, _D), bf16),
        ],
    )(x, m_bc, n0, n1)

    nblk = _TNQ // _HD
    qkv = pl.pallas_call(
        _proj_kernel,
        grid=(_B, _S // _TSQ, 3 * _D // _TNQ),
        in_specs=[
            pl.BlockSpec((1, _TSQ, _D), lambda b, i, n: (b, i, 0)),
            pl.BlockSpec((1, _TSQ, _D), lambda b, i, n: (b, i, 0)),
            pl.BlockSpec((_TNQ, _D), lambda b, i, n: (n, 0)),
            pl.BlockSpec((_TNQ, _D), lambda b, i, n: (n, 0)),
            pl.BlockSpec((1, _TSQ, _HD), lambda b, i, n: (n // 4, i, 0)),
            pl.BlockSpec((1, _TSQ, _HD), lambda b, i, n: (n // 4, i, 0)),
        ],
        out_specs=pl.BlockSpec((1, _TSQ, _TNQ), lambda b, i, n: (b, i, n)),
        out_shape=jax.ShapeDtypeStruct((_B, _S, 3 * _D), bf16),
    )(g0, g1, wqkv0, wqkv1, ca, sb)

    nq = _S // _TQ
    tri = [(i, j) for i in range(nq) for j in range(i + 1)]
    itab = jnp.array([i for i, _ in tri], dtype=jnp.int32)
    jtab = jnp.array([j for _, j in tri], dtype=jnp.int32)
    attn = pl.pallas_call(
        _attn_kernel,
        grid_spec=pltpu.PrefetchScalarGridSpec(
            num_scalar_prefetch=2,
            grid=(_B, _H, len(tri)),
            in_specs=[
                pl.BlockSpec((1, _TQ, _HD),
                             lambda b, h, t, it, jt: (b, it[t], h)),
                pl.BlockSpec((1, _TQ, _HD),
                             lambda b, h, t, it, jt: (b, jt[t], _H + h)),
                pl.BlockSpec((1, _TQ, _HD),
                             lambda b, h, t, it, jt: (b, jt[t], 2 * _H + h)),
            ],
            out_specs=pl.BlockSpec((1, _TQ, _HD),
                                   lambda b, h, t, it, jt: (b, it[t], h)),
            scratch_shapes=[
                pltpu.VMEM((_TQ, _HD), jnp.float32),
                pltpu.VMEM((_TQ, 128), jnp.float32),
                pltpu.VMEM((_TQ, 128), jnp.float32),
            ],
        ),
        out_shape=jax.ShapeDtypeStruct((_B, _S, _D), bf16),
    )(itab, jtab, qkv, qkv, qkv)

    out = pl.pallas_call(
        _out_kernel,
        grid=(_B, _S // _TSO, _D // _TN),
        in_specs=[
            pl.BlockSpec((1, _TSO, _D), lambda b, i, n: (b, i, 0)),
            pl.BlockSpec((1, _TSO, 128), lambda b, i, n: (b, i, 0)),
            pl.BlockSpec((_TN, _D), lambda b, i, n: (n, 0)),
            pl.BlockSpec((_TN, _D), lambda b, i, n: (n, 0)),
        ],
        out_specs=pl.BlockSpec((1, _TSO, _TN), lambda b, i, n: (b, i, n)),
        out_shape=jax.ShapeDtypeStruct((_B, _S, _D), jnp.float32),
    )(attn, m_bc, wo0, wo1)

    return out


# speedup vs baseline: 1.0655x; 1.0084x over previous
import jax
import jax.numpy as jnp
from jax.experimental import pallas as pl
from jax.experimental.pallas import tpu as pltpu

_B, _S, _D, _H = 2, 2048, 2048, 16
_HD = _D // _H
_EPS = 1e-5
_SCALE = 1.0 / (_HD ** 0.5)
_NEG = -1e9
_TSN = 512
_TSQ = 2048
_TNQ = 512
_TQ = 1024
_TSO = 1024
_TN = 512
_NT = (((1,), (1,)), ((), ()))
_NN = (((1,), (0,)), ((), ()))


def _norm_kernel(x_ref, m_ref, n0_ref, n1_ref, g0_ref, g1_ref):
    x = x_ref[0]
    r = jax.lax.rsqrt(jnp.mean(x * x, axis=1, keepdims=True) + _EPS)
    m1 = jnp.max(m_ref[0], axis=1, keepdims=True)
    h0 = x * (r * (1.0 - m1))
    h1 = x * (r * m1)
    g0_ref[0] = (h0 * n0_ref[...]).astype(jnp.bfloat16)
    g1_ref[0] = (h1 * n1_ref[...]).astype(jnp.bfloat16)


def _proj_kernel(g0_ref, g1_ref, w0_ref, w1_ref, ca_ref, sb_ref, o_ref):
    g0 = g0_ref[0]
    g1 = g1_ref[0]
    y = (jax.lax.dot_general(g0, w0_ref[...], _NT,
                             preferred_element_type=jnp.float32)
         + jax.lax.dot_general(g1, w1_ref[...], _NT,
                               preferred_element_type=jnp.float32))
    ia = jax.lax.broadcasted_iota(jnp.int32, (_TNQ, _TNQ), 0)
    ib = jax.lax.broadcasted_iota(jnp.int32, (_TNQ, _TNQ), 1)
    pmat = ((ia ^ 1) == ib).astype(jnp.bfloat16)
    sw = jax.lax.dot_general(y.astype(jnp.bfloat16), pmat, _NN,
                             preferred_element_type=jnp.float32)
    ca = jnp.concatenate([ca_ref[0]] * (_TNQ // _HD), axis=1)
    sb = jnp.concatenate([sb_ref[0]] * (_TNQ // _HD), axis=1)
    o_ref[0] = (y * ca + sw * sb).astype(jnp.bfloat16)


def _attn_kernel(it_ref, jt_ref, q_ref, k_ref, v_ref, o_ref,
                 acc_ref, m_ref, l_ref):
    t = pl.program_id(2)
    i = it_ref[t]
    j = jt_ref[t]
    q = q_ref[0]
    k = k_ref[0]
    s = jax.lax.dot_general(q, k, _NT, preferred_element_type=jnp.float32)
    row = jax.lax.broadcasted_iota(jnp.int32, (_TQ, _TQ), 0)
    col = jax.lax.broadcasted_iota(jnp.int32, (_TQ, _TQ), 1)
    s = jnp.where(jnp.logical_and(i == j, col > row), _NEG, s)
    first = j == 0
    m_prev = jnp.where(first, -3e38,
                       jnp.max(m_ref[...], axis=1, keepdims=True))
    l_prev = jnp.where(first, 0.0,
                       jnp.max(l_ref[...], axis=1, keepdims=True))
    acc_prev = jnp.where(first, 0.0, acc_ref[...])
    m_new = jnp.maximum(m_prev, jnp.max(s, axis=1, keepdims=True))
    p = jnp.exp(s - m_new)
    corr = jnp.exp(m_prev - m_new)
    l_new = l_prev * corr + jnp.sum(p, axis=1, keepdims=True)
    pv = jax.lax.dot_general(p.astype(jnp.bfloat16), v_ref[0], _NN,
                             preferred_element_type=jnp.float32)
    acc_new = acc_prev * corr + pv
    m_ref[...] = jnp.broadcast_to(m_new, (_TQ, 128))
    l_ref[...] = jnp.broadcast_to(l_new, (_TQ, 128))
    acc_ref[...] = acc_new

    @pl.when(i == j)
    def _():
        o_ref[0] = (acc_new / l_new).astype(jnp.bfloat16)


def _out_kernel(a_ref, m_ref, wo0_ref, wo1_ref, o_ref):
    a = a_ref[0]
    m1 = jnp.max(m_ref[0], axis=1, keepdims=True)
    a1 = a * m1.astype(jnp.bfloat16)
    a0 = a - a1
    w0 = wo0_ref[...].astype(jnp.bfloat16)
    w1 = wo1_ref[...].astype(jnp.bfloat16)
    o_ref[0] = (jax.lax.dot_general(a0, w0, _NT,
                                    preferred_element_type=jnp.float32)
                + jax.lax.dot_general(a1, w1, _NT,
                                      preferred_element_type=jnp.float32))


def kernel(x, wq0, wk0, wv0, wo0, wq1, wk1, wv1, wo1, nrm0, nrm1,
           freqs_cos, freqs_sin, mask, modality_ids, start_pos):
    del mask, start_pos
    f32 = jnp.float32
    bf16 = jnp.bfloat16
    m_bc = jnp.broadcast_to(
        (modality_ids == 1).astype(f32)[:, :, None], (_B, _S, 128))
    cc = jnp.repeat(freqs_cos, 2, axis=1)
    sign = jnp.tile(jnp.array([-1.0, 1.0], dtype=f32), _HD // 2)
    ss = jnp.repeat(freqs_sin, 2, axis=1) * sign[None, :]
    ca = jnp.stack([cc * _SCALE, cc, jnp.ones_like(cc)])
    sb = jnp.stack([ss * _SCALE, ss, jnp.zeros_like(ss)])
    n0 = nrm0.reshape(1, _D)
    n1 = nrm1.reshape(1, _D)
    wqkv0 = jnp.concatenate([wq0, wk0, wv0], axis=0).astype(bf16)
    wqkv1 = jnp.concatenate([wq1, wk1, wv1], axis=0).astype(bf16)

    g0, g1 = pl.pallas_call(
        _norm_kernel,
        grid=(_B, _S // _TSN),
        in_specs=[
            pl.BlockSpec((1, _TSN, _D), lambda b, i: (b, i, 0)),
            pl.BlockSpec((1, _TSN, 128), lambda b, i: (b, i, 0)),
            pl.BlockSpec((1, _D), lambda b, i: (0, 0)),
            pl.BlockSpec((1, _D), lambda b, i: (0, 0)),
        ],
        out_specs=[
            pl.BlockSpec((1, _TSN, _D), lambda b, i: (b, i, 0)),
            pl.BlockSpec((1, _TSN, _D), lambda b, i: (b, i, 0)),
        ],
        out_shape=[
            jax.ShapeDtypeStruct((_B, _S, _D), bf16),
            jax.ShapeDtypeStruct((_B, _S, _D), bf16),
        ],
    )(x, m_bc, n0, n1)

    qkv = pl.pallas_call(
        _proj_kernel,
        grid=(_B, _S // _TSQ, 3 * _D // _TNQ),
        in_specs=[
            pl.BlockSpec((1, _TSQ, _D), lambda b, i, n: (b, i, 0)),
            pl.BlockSpec((1, _TSQ, _D), lambda b, i, n: (b, i, 0)),
            pl.BlockSpec((_TNQ, _D), lambda b, i, n: (n, 0)),
            pl.BlockSpec((_TNQ, _D), lambda b, i, n: (n, 0)),
            pl.BlockSpec((1, _TSQ, _HD), lambda b, i, n: (n // 4, i, 0)),
            pl.BlockSpec((1, _TSQ, _HD), lambda b, i, n: (n // 4, i, 0)),
        ],
        out_specs=pl.BlockSpec((1, _TSQ, _TNQ), lambda b, i, n: (b, i, n)),
        out_shape=jax.ShapeDtypeStruct((_B, _S, 3 * _D), bf16),
    )(g0, g1, wqkv0, wqkv1, ca, sb)

    nq = _S // _TQ
    tri = [(i, j) for i in range(nq) for j in range(i + 1)]
    itab = jnp.array([i for i, _ in tri], dtype=jnp.int32)
    jtab = jnp.array([j for _, j in tri], dtype=jnp.int32)
    attn = pl.pallas_call(
        _attn_kernel,
        grid_spec=pltpu.PrefetchScalarGridSpec(
            num_scalar_prefetch=2,
            grid=(_B, _H, len(tri)),
            in_specs=[
                pl.BlockSpec((1, _TQ, _HD),
                             lambda b, h, t, it, jt: (b, it[t], h)),
                pl.BlockSpec((1, _TQ, _HD),
                             lambda b, h, t, it, jt: (b, jt[t], _H + h)),
                pl.BlockSpec((1, _TQ, _HD),
                             lambda b, h, t, it, jt: (b, jt[t], 2 * _H + h)),
            ],
            out_specs=pl.BlockSpec((1, _TQ, _HD),
                                   lambda b, h, t, it, jt: (b, it[t], h)),
            scratch_shapes=[
                pltpu.VMEM((_TQ, _HD), jnp.float32),
                pltpu.VMEM((_TQ, 128), jnp.float32),
                pltpu.VMEM((_TQ, 128), jnp.float32),
            ],
        ),
        out_shape=jax.ShapeDtypeStruct((_B, _S, _D), bf16),
    )(itab, jtab, qkv, qkv, qkv)

    out = pl.pallas_call(
        _out_kernel,
        grid=(_B, _S // _TSO, _D // _TN),
        in_specs=[
            pl.BlockSpec((1, _TSO, _D), lambda b, i, n: (b, i, 0)),
            pl.BlockSpec((1, _TSO, 128), lambda b, i, n: (b, i, 0)),
            pl.BlockSpec((_TN, _D), lambda b, i, n: (n, 0)),
            pl.BlockSpec((_TN, _D), lambda b, i, n: (n, 0)),
        ],
        out_specs=pl.BlockSpec((1, _TSO, _TN), lambda b, i, n: (b, i, n)),
        out_shape=jax.ShapeDtypeStruct((_B, _S, _D), jnp.float32),
    )(attn, m_bc, wo0, wo1)

    return out
